# baseline (speedup 1.0000x reference)
import functools

import jax
import jax.numpy as jnp
from jax import lax
from jax.experimental import pallas as pl
from jax.experimental.pallas import tpu as pltpu
from jax.experimental.pallas import tpu_sc as plsc

_N = 10000
_E = 320000
_DF = 128
_G = 16
_LAT = 64
_EPS = 1e-5

_NC = 2
_NS = 16
_NW = _NC * _NS
_CW = 128
_CPW = 80
_EP = _NW * _CPW * _CW
_NP = 10112
_RPT = _NP // _NS
_K = 8
_LAG = 4


def _conv_impl(g, src2d, dst2d, zrows, out, src_all, dst_all, rows, acc,
               *sems):
    gsem = sems[:_K]
    ssem = sems[_K:]
    cid = lax.axis_index("c")
    sid = lax.axis_index("s")
    wid = cid * _NS + sid
    pltpu.sync_copy(src2d.at[pl.ds(wid * _CPW, _CPW)], src_all)
    pltpu.sync_copy(dst2d.at[pl.ds(wid * _CPW, _CPW)], dst_all)
    pltpu.sync_copy(zrows.at[pl.ds(sid * _RPT, _RPT)],
                    acc.at[pl.ds(sid * _RPT, _RPT)])
    plsc.subcore_barrier()

    for b in range(_LAG):
        pltpu.async_copy(g.at[src_all.at[b]], rows.at[b], gsem[b])

    def body(i, carry):
        for u in range(_K):
            c = i * _K + u
            b = u
            pltpu.make_async_copy(g.at[src_all.at[c]], rows.at[b],
                                  gsem[b]).wait()
            pltpu.async_copy(rows.at[b], acc.at[dst_all.at[c]], ssem[b],
                             add=True)
            d = c + _LAG
            b2 = (u + _LAG) % _K

            @pl.when(d < _CPW)
            def _():
                @pl.when(d >= _K)
                def _():
                    pltpu.make_async_copy(rows.at[b2], acc.at[dst_all.at[c]],
                                          ssem[b2]).wait()
                pltpu.async_copy(g.at[src_all.at[d]], rows.at[b2], gsem[b2])
        return carry

    lax.fori_loop(0, _CPW // _K, body, 0)
    for b in range(_K):
        pltpu.make_async_copy(rows.at[b], acc.at[dst_all.at[0]],
                              ssem[b]).wait()
    plsc.subcore_barrier()
    pltpu.sync_copy(acc.at[pl.ds(sid * _RPT, _RPT)],
                    out.at[cid, pl.ds(sid * _RPT, _RPT)])


def _make_conv(feat):
    mesh = plsc.VectorSubcoreMesh(core_axis_name="c", subcore_axis_name="s")
    return pl.kernel(
        _conv_impl,
        out_type=jax.ShapeDtypeStruct((_NC, _NP, feat), jnp.float32),
        mesh=mesh,
        compiler_params=pltpu.CompilerParams(use_tc_tiling_on_sc=False),
        scratch_types=[
            pltpu.VMEM((_CPW, _CW), jnp.int32),
            pltpu.VMEM((_CPW, _CW), jnp.int32),
            pltpu.VMEM((_K, _CW, feat), jnp.float32),
            pltpu.VMEM_SHARED((_NP, feat), jnp.float32),
        ] + [pltpu.SemaphoreType.DMA] * (2 * _K),
    )


def _dense1_impl(degp_ref, x_ref, w1_ref, g1_ref, dinv_ref):
    degp = degp_ref[...]
    deg = degp[0, :_N, :1] + degp[1, :_N, :1] + 1.0
    dinv = lax.rsqrt(deg)
    h = jnp.dot(x_ref[...], w1_ref[...], preferred_element_type=jnp.float32)
    g1_ref[...] = h * dinv
    dinv_ref[...] = dinv


def _dense2_impl(accp_ref, g1_ref, dinv_ref, b1_ref, gamma_ref, beta_ref,
                 w2_ref, g2_ref):
    ap = accp_ref[...]
    dinv = dinv_ref[...]
    h = dinv * (ap[0, :_N] + ap[1, :_N] + g1_ref[...]) + b1_ref[...]
    mean = jnp.mean(h, axis=0, keepdims=True)
    c = h - mean
    var = jnp.mean(c * c, axis=0, keepdims=True)
    h = c * lax.rsqrt(var + _EPS) * gamma_ref[...] + beta_ref[...]
    h = jnp.maximum(h, 0.0)
    g2_ref[...] = jnp.dot(h, w2_ref[...],
                          preferred_element_type=jnp.float32) * dinv


def _dense3_impl(accp_ref, g2_ref, dinv_ref, b2_ref, batch_ref, wf1_ref,
                 bf1_ref, wf2_ref, bf2_ref, mu_ref, ls_ref):
    ap = accp_ref[...]
    out2 = dinv_ref[...] * (ap[0, :_N] + ap[1, :_N] + g2_ref[...]) + b2_ref[...]
    b = batch_ref[...]
    oh = (lax.broadcasted_iota(jnp.int32, (_G, _N), 0)
          == b[None, :]).astype(jnp.float32)
    counts = jnp.sum(oh, axis=1, keepdims=True)
    pooled = jnp.dot(oh, out2, preferred_element_type=jnp.float32)
    pooled = pooled / jnp.maximum(counts, 1.0)
    h = jnp.dot(pooled, wf1_ref[...], preferred_element_type=jnp.float32)
    h = jnp.maximum(h + bf1_ref[...], 0.0)
    o = jnp.dot(h, wf2_ref[...], preferred_element_type=jnp.float32)
    o = o + bf2_ref[...]
    mu_ref[...] = o[:, :_LAT]
    ls_ref[...] = o[:, _LAT:]


def _tc_call(body, out_shapes):
    return pl.pallas_call(body, out_shape=out_shapes)


@jax.jit
def kernel(x, edge_index, batch, W1, b1, gamma, beta, W2, b2, Wf1, bf1,
           Wf2, bf2):
    pad = _EP - _E
    pad_src = jnp.arange(pad, dtype=jnp.int32) % _N
    pad_dst = _N + (jnp.arange(pad, dtype=jnp.int32) % (_NP - _N))
    src2d = jnp.concatenate(
        [edge_index[0], pad_src]).reshape(_EP // _CW, _CW)
    dst2d = jnp.concatenate(
        [edge_index[1], pad_dst]).reshape(_EP // _CW, _CW)
    z32 = jnp.zeros((_NP, 32), jnp.float32)
    z64 = jnp.zeros((_NP, 64), jnp.float32)

    ones_np = jnp.ones((_NP, 32), jnp.float32)
    degp = _make_conv(32)(ones_np, dst2d, dst2d, z32)

    g1, dinv = _tc_call(
        _dense1_impl,
        (jax.ShapeDtypeStruct((_N, 32), jnp.float32),
         jax.ShapeDtypeStruct((_N, 1), jnp.float32)),
    )(degp, x, W1)

    acc1 = _make_conv(32)(g1, src2d, dst2d, z32)

    g2 = _tc_call(
        _dense2_impl,
        jax.ShapeDtypeStruct((_N, 64), jnp.float32),
    )(acc1, g1, dinv, b1.reshape(1, 32), gamma.reshape(1, 32),
      beta.reshape(1, 32), W2)

    acc2 = _make_conv(64)(g2, src2d, dst2d, z64)

    mu, ls = _tc_call(
        _dense3_impl,
        (jax.ShapeDtypeStruct((_G, _LAT), jnp.float32),
         jax.ShapeDtypeStruct((_G, _LAT), jnp.float32)),
    )(acc2, g2, dinv, b2.reshape(1, 64), batch, Wf1, bf1.reshape(1, 128),
      Wf2, bf2.reshape(1, 2 * _LAT))

    return (mu, ls)


# speedup vs baseline: 1.0580x; 1.0580x over previous
import functools

import jax
import jax.numpy as jnp
from jax import lax
from jax.experimental import pallas as pl
from jax.experimental.pallas import tpu as pltpu
from jax.experimental.pallas import tpu_sc as plsc

_N = 10000
_E = 320000
_DF = 128
_G = 16
_LAT = 64
_EPS = 1e-5

_NC = 2
_NS = 16
_NW = _NC * _NS
_CW = 128
_CPW = 80
_EP = _NW * _CPW * _CW
_NP = 10112
_RPT = _NP // _NS
_K = 8
_LAG = 4


def _conv_impl(g, src2d, dst2d, zrows, out, src_all, dst_all, rows, acc,
               *sems):
    gsem = sems
    cid = lax.axis_index("c")
    sid = lax.axis_index("s")
    wid = cid * _NS + sid
    pltpu.sync_copy(src2d.at[pl.ds(wid * _CPW, _CPW)], src_all)
    pltpu.sync_copy(dst2d.at[pl.ds(wid * _CPW, _CPW)], dst_all)
    pltpu.sync_copy(zrows.at[pl.ds(sid * _RPT, _RPT)],
                    acc.at[pl.ds(sid * _RPT, _RPT)])
    plsc.subcore_barrier()

    for b in range(_K):
        pltpu.async_copy(g.at[src_all.at[b]], rows.at[b], gsem[b])

    def body(i, carry):
        for u in range(_K):
            c = i * _K + u
            b = u
            pltpu.make_async_copy(g.at[src_all.at[c]], rows.at[b],
                                  gsem[b]).wait()
            pltpu.sync_copy(rows.at[b], acc.at[dst_all.at[c]], add=True)

            @pl.when(c + _K < _CPW)
            def _():
                pltpu.async_copy(g.at[src_all.at[c + _K]], rows.at[b],
                                 gsem[b])
        return carry

    lax.fori_loop(0, _CPW // _K, body, 0)
    plsc.subcore_barrier()
    pltpu.sync_copy(acc.at[pl.ds(sid * _RPT, _RPT)],
                    out.at[cid, pl.ds(sid * _RPT, _RPT)])


def _make_conv(feat):
    mesh = plsc.VectorSubcoreMesh(core_axis_name="c", subcore_axis_name="s")
    return pl.kernel(
        _conv_impl,
        out_type=jax.ShapeDtypeStruct((_NC, _NP, feat), jnp.float32),
        mesh=mesh,
        compiler_params=pltpu.CompilerParams(use_tc_tiling_on_sc=False),
        scratch_types=[
            pltpu.VMEM((_CPW, _CW), jnp.int32),
            pltpu.VMEM((_CPW, _CW), jnp.int32),
            pltpu.VMEM((_K, _CW, feat), jnp.float32),
            pltpu.VMEM_SHARED((_NP, feat), jnp.float32),
        ] + [pltpu.SemaphoreType.DMA] * _K,
    )


def _dense1_impl(degp_ref, x_ref, w1_ref, g1_ref, dinv_ref):
    degp = degp_ref[...]
    deg = degp[0, :_N, :1] + degp[1, :_N, :1] + 1.0
    dinv = lax.rsqrt(deg)
    h = jnp.dot(x_ref[...], w1_ref[...], preferred_element_type=jnp.float32)
    g1_ref[...] = h * dinv
    dinv_ref[...] = dinv


def _dense2_impl(accp_ref, g1_ref, dinv_ref, b1_ref, gamma_ref, beta_ref,
                 w2_ref, g2_ref):
    ap = accp_ref[...]
    dinv = dinv_ref[...]
    h = dinv * (ap[0, :_N] + ap[1, :_N] + g1_ref[...]) + b1_ref[...]
    mean = jnp.mean(h, axis=0, keepdims=True)
    c = h - mean
    var = jnp.mean(c * c, axis=0, keepdims=True)
    h = c * lax.rsqrt(var + _EPS) * gamma_ref[...] + beta_ref[...]
    h = jnp.maximum(h, 0.0)
    g2_ref[...] = jnp.dot(h, w2_ref[...],
                          preferred_element_type=jnp.float32) * dinv


def _dense3_impl(accp_ref, g2_ref, dinv_ref, b2_ref, batch_ref, wf1_ref,
                 bf1_ref, wf2_ref, bf2_ref, mu_ref, ls_ref):
    ap = accp_ref[...]
    out2 = dinv_ref[...] * (ap[0, :_N] + ap[1, :_N] + g2_ref[...]) + b2_ref[...]
    b = batch_ref[...]
    oh = (lax.broadcasted_iota(jnp.int32, (_G, _N), 0)
          == b[None, :]).astype(jnp.float32)
    counts = jnp.sum(oh, axis=1, keepdims=True)
    pooled = jnp.dot(oh, out2, preferred_element_type=jnp.float32)
    pooled = pooled / jnp.maximum(counts, 1.0)
    h = jnp.dot(pooled, wf1_ref[...], preferred_element_type=jnp.float32)
    h = jnp.maximum(h + bf1_ref[...], 0.0)
    o = jnp.dot(h, wf2_ref[...], preferred_element_type=jnp.float32)
    o = o + bf2_ref[...]
    mu_ref[...] = o[:, :_LAT]
    ls_ref[...] = o[:, _LAT:]


def _tc_call(body, out_shapes):
    return pl.pallas_call(body, out_shape=out_shapes)


@jax.jit
def kernel(x, edge_index, batch, W1, b1, gamma, beta, W2, b2, Wf1, bf1,
           Wf2, bf2):
    pad = _EP - _E
    pad_src = jnp.arange(pad, dtype=jnp.int32) % _N
    pad_dst = _N + (jnp.arange(pad, dtype=jnp.int32) % (_NP - _N))
    src2d = jnp.concatenate(
        [edge_index[0], pad_src]).reshape(_EP // _CW, _CW)
    dst2d = jnp.concatenate(
        [edge_index[1], pad_dst]).reshape(_EP // _CW, _CW)
    z32 = jnp.zeros((_NP, 32), jnp.float32)
    z64 = jnp.zeros((_NP, 64), jnp.float32)

    ones_np = jnp.ones((_NP, 32), jnp.float32)
    degp = _make_conv(32)(ones_np, dst2d, dst2d, z32)

    g1, dinv = _tc_call(
        _dense1_impl,
        (jax.ShapeDtypeStruct((_N, 32), jnp.float32),
         jax.ShapeDtypeStruct((_N, 1), jnp.float32)),
    )(degp, x, W1)

    acc1 = _make_conv(32)(g1, src2d, dst2d, z32)

    g2 = _tc_call(
        _dense2_impl,
        jax.ShapeDtypeStruct((_N, 64), jnp.float32),
    )(acc1, g1, dinv, b1.reshape(1, 32), gamma.reshape(1, 32),
      beta.reshape(1, 32), W2)

    acc2 = _make_conv(64)(g2, src2d, dst2d, z64)

    mu, ls = _tc_call(
        _dense3_impl,
        (jax.ShapeDtypeStruct((_G, _LAT), jnp.float32),
         jax.ShapeDtypeStruct((_G, _LAT), jnp.float32)),
    )(acc2, g2, dinv, b2.reshape(1, 64), batch, Wf1, bf1.reshape(1, 128),
      Wf2, bf2.reshape(1, 2 * _LAT))

    return (mu, ls)


# speedup vs baseline: 1.0839x; 1.0245x over previous
import functools

import jax
import jax.numpy as jnp
from jax import lax
from jax.experimental import pallas as pl
from jax.experimental.pallas import tpu as pltpu
from jax.experimental.pallas import tpu_sc as plsc

_N = 10000
_E = 320000
_DF = 128
_G = 16
_LAT = 64
_EPS = 1e-5

_NC = 2
_NS = 16
_NW = _NC * _NS
_CW = 128
_CPW = 80
_EP = _NW * _CPW * _CW
_NP = 10112
_RPT = _NP // _NS
_K = 8
_LAG = 4


def _deg_impl(dst2d, ones_c, zrows, out, dst_all, ones_v, acc):
    cid = lax.axis_index("c")
    sid = lax.axis_index("s")
    wid = cid * _NS + sid
    pltpu.sync_copy(ones_c, ones_v)
    pltpu.sync_copy(dst2d.at[pl.ds(wid * _CPW, _CPW)], dst_all)
    pltpu.sync_copy(zrows.at[pl.ds(sid * _RPT, _RPT)],
                    acc.at[pl.ds(sid * _RPT, _RPT)])
    plsc.subcore_barrier()

    def body(j, carry):
        pltpu.sync_copy(ones_v, acc.at[dst_all.at[j]], add=True)
        return carry

    lax.fori_loop(0, _CPW, body, 0)
    plsc.subcore_barrier()
    pltpu.sync_copy(acc.at[pl.ds(sid * _RPT, _RPT)],
                    out.at[cid, pl.ds(sid * _RPT, _RPT)])


def _make_deg():
    mesh = plsc.VectorSubcoreMesh(core_axis_name="c", subcore_axis_name="s")
    return pl.kernel(
        _deg_impl,
        out_type=jax.ShapeDtypeStruct((_NC, _NP, 32), jnp.float32),
        mesh=mesh,
        compiler_params=pltpu.CompilerParams(use_tc_tiling_on_sc=False),
        scratch_types=[
            pltpu.VMEM((_CPW, _CW), jnp.int32),
            pltpu.VMEM((_CW, 32), jnp.float32),
            pltpu.VMEM_SHARED((_NP, 32), jnp.float32),
        ],
    )


def _conv_impl(g, src2d, dst2d, zrows, out, src_all, dst_all, rows, acc,
               *sems):
    gsem = sems
    cid = lax.axis_index("c")
    sid = lax.axis_index("s")
    wid = cid * _NS + sid
    pltpu.sync_copy(src2d.at[pl.ds(wid * _CPW, _CPW)], src_all)
    pltpu.sync_copy(dst2d.at[pl.ds(wid * _CPW, _CPW)], dst_all)
    pltpu.sync_copy(zrows.at[pl.ds(sid * _RPT, _RPT)],
                    acc.at[pl.ds(sid * _RPT, _RPT)])
    plsc.subcore_barrier()

    for b in range(_K):
        pltpu.async_copy(g.at[src_all.at[b]], rows.at[b], gsem[b])

    def body(i, carry):
        for u in range(_K):
            c = i * _K + u
            b = u
            pltpu.make_async_copy(g.at[src_all.at[c]], rows.at[b],
                                  gsem[b]).wait()
            pltpu.sync_copy(rows.at[b], acc.at[dst_all.at[c]], add=True)

            @pl.when(c + _K < _CPW)
            def _():
                pltpu.async_copy(g.at[src_all.at[c + _K]], rows.at[b],
                                 gsem[b])
        return carry

    lax.fori_loop(0, _CPW // _K, body, 0)
    plsc.subcore_barrier()
    pltpu.sync_copy(acc.at[pl.ds(sid * _RPT, _RPT)],
                    out.at[cid, pl.ds(sid * _RPT, _RPT)])


def _make_conv(feat):
    mesh = plsc.VectorSubcoreMesh(core_axis_name="c", subcore_axis_name="s")
    return pl.kernel(
        _conv_impl,
        out_type=jax.ShapeDtypeStruct((_NC, _NP, feat), jnp.float32),
        mesh=mesh,
        compiler_params=pltpu.CompilerParams(use_tc_tiling_on_sc=False),
        scratch_types=[
            pltpu.VMEM((_CPW, _CW), jnp.int32),
            pltpu.VMEM((_CPW, _CW), jnp.int32),
            pltpu.VMEM((_K, _CW, feat), jnp.float32),
            pltpu.VMEM_SHARED((_NP, feat), jnp.float32),
        ] + [pltpu.SemaphoreType.DMA] * _K,
    )


def _dense1_impl(degp_ref, x_ref, w1_ref, g1_ref, dinv_ref):
    degp = degp_ref[...]
    deg = degp[0, :_N, :1] + degp[1, :_N, :1] + 1.0
    dinv = lax.rsqrt(deg)
    h = jnp.dot(x_ref[...], w1_ref[...], preferred_element_type=jnp.float32)
    g1_ref[...] = h * dinv
    dinv_ref[...] = dinv


def _dense2_impl(accp_ref, g1_ref, dinv_ref, b1_ref, gamma_ref, beta_ref,
                 w2_ref, g2_ref):
    ap = accp_ref[...]
    dinv = dinv_ref[...]
    h = dinv * (ap[0, :_N] + ap[1, :_N] + g1_ref[...]) + b1_ref[...]
    mean = jnp.mean(h, axis=0, keepdims=True)
    c = h - mean
    var = jnp.mean(c * c, axis=0, keepdims=True)
    h = c * lax.rsqrt(var + _EPS) * gamma_ref[...] + beta_ref[...]
    h = jnp.maximum(h, 0.0)
    g2_ref[...] = jnp.dot(h, w2_ref[...],
                          preferred_element_type=jnp.float32) * dinv


def _dense3_impl(accp_ref, g2_ref, dinv_ref, b2_ref, batch_ref, wf1_ref,
                 bf1_ref, wf2_ref, bf2_ref, mu_ref, ls_ref):
    ap = accp_ref[...]
    out2 = dinv_ref[...] * (ap[0, :_N] + ap[1, :_N] + g2_ref[...]) + b2_ref[...]
    b = batch_ref[...]
    oh = (lax.broadcasted_iota(jnp.int32, (_G, _N), 0)
          == b[None, :]).astype(jnp.float32)
    counts = jnp.sum(oh, axis=1, keepdims=True)
    pooled = jnp.dot(oh, out2, preferred_element_type=jnp.float32)
    pooled = pooled / jnp.maximum(counts, 1.0)
    h = jnp.dot(pooled, wf1_ref[...], preferred_element_type=jnp.float32)
    h = jnp.maximum(h + bf1_ref[...], 0.0)
    o = jnp.dot(h, wf2_ref[...], preferred_element_type=jnp.float32)
    o = o + bf2_ref[...]
    mu_ref[...] = o[:, :_LAT]
    ls_ref[...] = o[:, _LAT:]


def _tc_call(body, out_shapes):
    return pl.pallas_call(body, out_shape=out_shapes)


@jax.jit
def kernel(x, edge_index, batch, W1, b1, gamma, beta, W2, b2, Wf1, bf1,
           Wf2, bf2):
    pad = _EP - _E
    pad_src = jnp.arange(pad, dtype=jnp.int32) % _N
    pad_dst = _N + (jnp.arange(pad, dtype=jnp.int32) % (_NP - _N))
    src2d = jnp.concatenate(
        [edge_index[0], pad_src]).reshape(_EP // _CW, _CW)
    dst2d = jnp.concatenate(
        [edge_index[1], pad_dst]).reshape(_EP // _CW, _CW)
    z32 = jnp.zeros((_NP, 32), jnp.float32)
    z64 = jnp.zeros((_NP, 64), jnp.float32)

    ones_c = jnp.ones((_CW, 32), jnp.float32)
    degp = _make_deg()(dst2d, ones_c, z32)

    g1, dinv = _tc_call(
        _dense1_impl,
        (jax.ShapeDtypeStruct((_N, 32), jnp.float32),
         jax.ShapeDtypeStruct((_N, 1), jnp.float32)),
    )(degp, x, W1)

    acc1 = _make_conv(32)(g1, src2d, dst2d, z32)

    g2 = _tc_call(
        _dense2_impl,
        jax.ShapeDtypeStruct((_N, 64), jnp.float32),
    )(acc1, g1, dinv, b1.reshape(1, 32), gamma.reshape(1, 32),
      beta.reshape(1, 32), W2)

    acc2 = _make_conv(64)(g2, src2d, dst2d, z64)

    mu, ls = _tc_call(
        _dense3_impl,
        (jax.ShapeDtypeStruct((_G, _LAT), jnp.float32),
         jax.ShapeDtypeStruct((_G, _LAT), jnp.float32)),
    )(acc2, g2, dinv, b2.reshape(1, 64), batch, Wf1, bf1.reshape(1, 128),
      Wf2, bf2.reshape(1, 2 * _LAT))

    return (mu, ls)


# speedup vs baseline: 1.0847x; 1.0007x over previous
import functools

import jax
import jax.numpy as jnp
from jax import lax
from jax.experimental import pallas as pl
from jax.experimental.pallas import tpu as pltpu
from jax.experimental.pallas import tpu_sc as plsc

_N = 10000
_E = 320000
_DF = 128
_G = 16
_LAT = 64
_EPS = 1e-5

_NC = 2
_NS = 16
_NW = _NC * _NS
_CW = 128
_CPW = 80
_EP = _NW * _CPW * _CW
_NP = 10112
_RPT = _NP // _NS
_K = 8


def _deg_impl(dst2d, ones_c, zrows, out, dst_all, ones_v, acc):
    cid = lax.axis_index("c")
    sid = lax.axis_index("s")
    wid = cid * _NS + sid
    pltpu.sync_copy(ones_c, ones_v)
    pltpu.sync_copy(dst2d.at[pl.ds(wid * _CPW, _CPW)], dst_all)
    pltpu.sync_copy(zrows.at[pl.ds(sid * _RPT, _RPT)],
                    acc.at[pl.ds(sid * _RPT, _RPT)])
    plsc.subcore_barrier()

    def body(j, carry):
        pltpu.sync_copy(ones_v, acc.at[dst_all.at[j]], add=True)
        return carry

    lax.fori_loop(0, _CPW, body, 0)
    plsc.subcore_barrier()
    pltpu.sync_copy(acc.at[pl.ds(sid * _RPT, _RPT)],
                    out.at[cid, pl.ds(sid * _RPT, _RPT)])


def _make_deg():
    mesh = plsc.VectorSubcoreMesh(core_axis_name="c", subcore_axis_name="s")
    return pl.kernel(
        _deg_impl,
        out_type=jax.ShapeDtypeStruct((_NC, _NP, 32), jnp.float32),
        mesh=mesh,
        compiler_params=pltpu.CompilerParams(use_tc_tiling_on_sc=False),
        scratch_types=[
            pltpu.VMEM((_CPW, _CW), jnp.int32),
            pltpu.VMEM((_CW, 32), jnp.float32),
            pltpu.VMEM_SHARED((_NP, 32), jnp.float32),
        ],
    )


def _conv_impl(g, src2d, dst2d, zrows, out, src_all, dst_all, rows, acc,
               *sems):
    gsem = sems
    cid = lax.axis_index("c")
    sid = lax.axis_index("s")
    wid = cid * _NS + sid
    pltpu.sync_copy(src2d.at[pl.ds(wid * _CPW, _CPW)], src_all)
    pltpu.sync_copy(dst2d.at[pl.ds(wid * _CPW, _CPW)], dst_all)
    pltpu.sync_copy(zrows.at[pl.ds(sid * _RPT, _RPT)],
                    acc.at[pl.ds(sid * _RPT, _RPT)])
    plsc.subcore_barrier()

    for b in range(_K):
        pltpu.async_copy(g.at[src_all.at[b]], rows.at[b], gsem[b])

    def body(i, carry):
        for u in range(_K):
            c = i * _K + u
            b = u
            pltpu.make_async_copy(g.at[src_all.at[c]], rows.at[b],
                                  gsem[b]).wait()
            pltpu.sync_copy(rows.at[b], acc.at[dst_all.at[c]], add=True)

            @pl.when(c + _K < _CPW)
            def _():
                pltpu.async_copy(g.at[src_all.at[c + _K]], rows.at[b],
                                 gsem[b])
        return carry

    lax.fori_loop(0, _CPW // _K, body, 0)
    plsc.subcore_barrier()
    pltpu.sync_copy(acc.at[pl.ds(sid * _RPT, _RPT)],
                    out.at[cid, pl.ds(sid * _RPT, _RPT)])


def _make_conv(feat):
    mesh = plsc.VectorSubcoreMesh(core_axis_name="c", subcore_axis_name="s")
    return pl.kernel(
        _conv_impl,
        out_type=jax.ShapeDtypeStruct((_NC, _NP, feat), jnp.float32),
        mesh=mesh,
        compiler_params=pltpu.CompilerParams(use_tc_tiling_on_sc=False),
        scratch_types=[
            pltpu.VMEM((_CPW, _CW), jnp.int32),
            pltpu.VMEM((_CPW, _CW), jnp.int32),
            pltpu.VMEM((_K, _CW, feat), jnp.float32),
            pltpu.VMEM_SHARED((_NP, feat), jnp.float32),
        ] + [pltpu.SemaphoreType.DMA] * _K,
    )


def _dense1_impl(degp_ref, x_ref, w1_ref, g1_ref, dinv_ref):
    degp = degp_ref[...]
    deg = degp[0, :_N, :1] + degp[1, :_N, :1] + 1.0
    dinv = lax.rsqrt(deg)
    h = jnp.dot(x_ref[...], w1_ref[...], preferred_element_type=jnp.float32)
    g1_ref[...] = h * dinv
    dinv_ref[...] = dinv


def _dense2_impl(accp_ref, g1_ref, dinv_ref, b1_ref, gamma_ref, beta_ref,
                 w2_ref, g2_ref):
    ap = accp_ref[...]
    dinv = dinv_ref[...]
    h = dinv * (ap[0, :_N] + ap[1, :_N] + g1_ref[...]) + b1_ref[...]
    mean = jnp.mean(h, axis=0, keepdims=True)
    c = h - mean
    var = jnp.mean(c * c, axis=0, keepdims=True)
    h = c * lax.rsqrt(var + _EPS) * gamma_ref[...] + beta_ref[...]
    h = jnp.maximum(h, 0.0)
    g2_ref[...] = jnp.dot(h, w2_ref[...],
                          preferred_element_type=jnp.float32) * dinv


def _dense3_impl(accp_ref, g2_ref, dinv_ref, b2_ref, batch_ref, wf1_ref,
                 bf1_ref, wf2_ref, bf2_ref, mu_ref, ls_ref):
    ap = accp_ref[...]
    out2 = dinv_ref[...] * (ap[0, :_N] + ap[1, :_N] + g2_ref[...]) + b2_ref[...]
    b = batch_ref[...]
    oh = (lax.broadcasted_iota(jnp.int32, (_G, _N), 0)
          == b[None, :]).astype(jnp.float32)
    counts = jnp.sum(oh, axis=1, keepdims=True)
    pooled = jnp.dot(oh, out2, preferred_element_type=jnp.float32)
    pooled = pooled / jnp.maximum(counts, 1.0)
    h = jnp.dot(pooled, wf1_ref[...], preferred_element_type=jnp.float32)
    h = jnp.maximum(h + bf1_ref[...], 0.0)
    o = jnp.dot(h, wf2_ref[...], preferred_element_type=jnp.float32)
    o = o + bf2_ref[...]
    mu_ref[...] = o[:, :_LAT]
    ls_ref[...] = o[:, _LAT:]


def _tc_call(body, out_shapes):
    return pl.pallas_call(body, out_shape=out_shapes)


@jax.jit
def kernel(x, edge_index, batch, W1, b1, gamma, beta, W2, b2, Wf1, bf1,
           Wf2, bf2):
    pad = _EP - _E
    pad_src = jnp.arange(pad, dtype=jnp.int32) % _N
    pad_dst = _N + (jnp.arange(pad, dtype=jnp.int32) % (_NP - _N))
    src2d = jnp.concatenate(
        [edge_index[0], pad_src]).reshape(_EP // _CW, _CW)
    dst2d = jnp.concatenate(
        [edge_index[1], pad_dst]).reshape(_EP // _CW, _CW)
    z32 = jnp.zeros((_NP, 32), jnp.float32)
    z64 = jnp.zeros((_NP, 64), jnp.float32)

    ones_c = jnp.ones((_CW, 32), jnp.float32)
    degp = _make_deg()(dst2d, ones_c, z32)

    g1, dinv = _tc_call(
        _dense1_impl,
        (jax.ShapeDtypeStruct((_N, 32), jnp.float32),
         jax.ShapeDtypeStruct((_N, 1), jnp.float32)),
    )(degp, x, W1)

    acc1 = _make_conv(32)(g1, src2d, dst2d, z32)

    g2 = _tc_call(
        _dense2_impl,
        jax.ShapeDtypeStruct((_N, 64), jnp.float32),
    )(acc1, g1, dinv, b1.reshape(1, 32), gamma.reshape(1, 32),
      beta.reshape(1, 32), W2)

    acc2 = _make_conv(64)(g2, src2d, dst2d, z64)

    mu, ls = _tc_call(
        _dense3_impl,
        (jax.ShapeDtypeStruct((_G, _LAT), jnp.float32),
         jax.ShapeDtypeStruct((_G, _LAT), jnp.float32)),
    )(acc2, g2, dinv, b2.reshape(1, 64), batch, Wf1, bf1.reshape(1, 128),
      Wf2, bf2.reshape(1, 2 * _LAT))

    return (mu, ls)
